# baseline (speedup 1.0000x reference)
import functools

import jax
import jax.numpy as jnp
from jax import lax
from jax.experimental import pallas as pl
from jax.experimental.pallas import tpu as pltpu
from jax.experimental.pallas import tpu_sc as plsc

NN = 100000
NP = 102400
EE = 1600000
EP = 1638400
GG = 64
CC = 32
NC, NS, LANES = 2, 16, 16
NW = NC * NS

BN = 2048
BE = 4096
GCH = 1024
SCH = 4096

_mesh = functools.partial(
    plsc.VectorSubcoreMesh,
    core_axis_name="c", subcore_axis_name="s",
    num_cores=NC, num_subcores=NS)


def _pre1_body(pos_ref, wA_ref, wB1_ref, wB2_ref, b_ref, a_ref, b1_ref, b2_ref):
    p = pos_ref[...]
    a_ref[...] = jnp.dot(p, wA_ref[...]) + b_ref[...]
    b1_ref[...] = jnp.dot(p, wB1_ref[...])
    b2_ref[...] = jnp.dot(p, wB2_ref[...])


def _pre1(pos, wA, wB1, wB2, b1a):
    grid = NP // BN
    return pl.pallas_call(
        _pre1_body,
        grid=(grid,),
        in_specs=[
            pl.BlockSpec((BN, 4), lambda i: (i, 0)),
            pl.BlockSpec((4, CC), lambda i: (0, 0)),
            pl.BlockSpec((4, CC), lambda i: (0, 0)),
            pl.BlockSpec((4, CC), lambda i: (0, 0)),
            pl.BlockSpec((1, CC), lambda i: (0, 0)),
        ],
        out_specs=[
            pl.BlockSpec((BN, CC), lambda i: (i, 0)),
            pl.BlockSpec((BN, CC), lambda i: (i, 0)),
            pl.BlockSpec((BN, CC), lambda i: (i, 0)),
        ],
        out_shape=[jax.ShapeDtypeStruct((NP, CC), jnp.float32)] * 3,
    )(pos, wA, wB1, wB2, b1a)


def _pre2_body(hT_ref, pos_ref, w32_ref, wd_ref, b_ref, a_ref):
    a = lax.dot_general(hT_ref[...], w32_ref[...], (((0,), (0,)), ((), ())))
    a_ref[...] = a + jnp.dot(pos_ref[...], wd_ref[...]) + b_ref[...]


def _pre2(hT, pos, w32, wd, b1b):
    grid = NP // BN
    return pl.pallas_call(
        _pre2_body,
        grid=(grid,),
        in_specs=[
            pl.BlockSpec((CC, BN), lambda i: (0, i)),
            pl.BlockSpec((BN, 4), lambda i: (i, 0)),
            pl.BlockSpec((CC, CC), lambda i: (0, 0)),
            pl.BlockSpec((4, CC), lambda i: (0, 0)),
            pl.BlockSpec((1, CC), lambda i: (0, 0)),
        ],
        out_specs=pl.BlockSpec((BN, CC), lambda i: (i, 0)),
        out_shape=jax.ShapeDtypeStruct((NP, CC), jnp.float32),
    )(hT, pos, w32, wd, b1b)


def _sc_gather_body(A_hbm, B_hbm, src_hbm, dst_hbm, gA_hbm, gB_hbm,
                    idxs_v, idxd_v, rowsA_v, rowsB_v, sem):
    wid = lax.axis_index("s") * NC + lax.axis_index("c")
    per_w = EP // NW
    base = wid * per_w

    def chunk(i, carry):
        off = base + i * GCH
        pltpu.sync_copy(src_hbm.at[pl.ds(off, GCH)], idxs_v)
        pltpu.sync_copy(dst_hbm.at[pl.ds(off, GCH)], idxd_v)
        cpA = pltpu.async_copy(A_hbm.at[idxs_v], rowsA_v, sem)
        cpB = pltpu.async_copy(B_hbm.at[idxd_v], rowsB_v, sem)
        cpA.wait()
        cpB.wait()
        pltpu.sync_copy(rowsA_v, gA_hbm.at[pl.ds(off, GCH)])
        pltpu.sync_copy(rowsB_v, gB_hbm.at[pl.ds(off, GCH)])
        return carry

    lax.fori_loop(0, per_w // GCH, chunk, 0)


def _sc_gather(A, B, src, dst):
    return pl.kernel(
        _sc_gather_body,
        compiler_params=pltpu.CompilerParams(use_tc_tiling_on_sc=False),
        out_type=[jax.ShapeDtypeStruct((EP, CC), jnp.float32),
                  jax.ShapeDtypeStruct((EP, CC), jnp.float32)],
        mesh=_mesh(),
        scratch_types=[
            pltpu.VMEM((GCH,), jnp.int32),
            pltpu.VMEM((GCH,), jnp.int32),
            pltpu.VMEM((GCH, CC), jnp.float32),
            pltpu.VMEM((GCH, CC), jnp.float32),
            pltpu.SemaphoreType.DMA,
        ],
    )(A, B, src, dst)


def _mlp_body(gA_ref, gB_ref, w2_ref, b2_ref, out_ref):
    pre = jnp.maximum(gA_ref[...] - gB_ref[...], 0.0)
    mt = lax.dot_general(w2_ref[...], pre, (((0,), (1,)), ((), ())))
    mt = mt + b2_ref[...]
    out_ref[...] = mt.reshape(CC, BE // 128, 128)


def _mlp(gA, gB, w2, b2):
    grid = EP // BE
    return pl.pallas_call(
        _mlp_body,
        grid=(grid,),
        in_specs=[
            pl.BlockSpec((BE, CC), lambda i: (i, 0)),
            pl.BlockSpec((BE, CC), lambda i: (i, 0)),
            pl.BlockSpec((CC, CC), lambda i: (0, 0)),
            pl.BlockSpec((CC, 1), lambda i: (0, 0)),
        ],
        out_specs=pl.BlockSpec((CC, BE // 128, 128), lambda i: (0, i, 0)),
        out_shape=jax.ShapeDtypeStruct((CC, EP // 128, 128), jnp.float32),
    )(gA, gB, w2, b2)


def _sc_scatter_body(mflat_hbm, dst_hbm, out_hbm, agg_v, midx_v, mval_v):
    wid = lax.axis_index("s") * NC + lax.axis_index("c")

    def zero(i, carry):
        agg_v[pl.ds(i * LANES, LANES)] = jnp.zeros((LANES,), jnp.float32)
        return carry

    lax.fori_loop(0, NP // LANES, zero, 0)

    def chunk(ci, carry):
        off = ci * SCH
        pltpu.sync_copy(dst_hbm.at[pl.ds(off, SCH)], midx_v)
        pltpu.sync_copy(mflat_hbm.at[pl.ds(wid * EP + off, SCH)], mval_v)

        def vec(v, cc):
            idx = midx_v[pl.ds(v * LANES, LANES)]
            val = mval_v[pl.ds(v * LANES, LANES)]
            cur = plsc.load_gather(agg_v, [idx])
            plsc.store_scatter(agg_v, [idx], jnp.maximum(cur, val))
            dupcnt, _ = plsc.scan_count(idx)

            def fix(_):
                def cond(n):
                    return jnp.any(n)

                def body(n):
                    plsc.store_scatter(agg_v, [idx], val, mask=n)
                    return val > plsc.load_gather(agg_v, [idx])

                need = val > plsc.load_gather(agg_v, [idx])
                lax.while_loop(cond, body, need)
                return 0

            lax.cond(jnp.any(dupcnt > 0), fix, lambda _: 0, 0)
            return cc

        lax.fori_loop(0, SCH // LANES, vec, 0)
        return carry

    lax.fori_loop(0, EP // SCH, chunk, 0)
    pltpu.sync_copy(agg_v, out_hbm.at[wid])


def _sc_scatter(mflat, dst):
    return pl.kernel(
        _sc_scatter_body,
        compiler_params=pltpu.CompilerParams(use_tc_tiling_on_sc=False,
                                             needs_layout_passes=False),
        out_type=jax.ShapeDtypeStruct((CC, NP), jnp.float32),
        mesh=_mesh(),
        scratch_types=[
            pltpu.VMEM((NP,), jnp.float32),
            pltpu.VMEM((SCH,), jnp.int32),
            pltpu.VMEM((SCH,), jnp.float32),
        ],
    )(mflat, dst)


def _pool_body(hT_ref, batch_ref, wl_ref, bl_ref, out_ref, acc_ref):
    i = pl.program_id(0)

    @pl.when(i == 0)
    def _():
        acc_ref[...] = jnp.zeros_like(acc_ref)

    h = hT_ref[...]
    bb = batch_ref[...].reshape(1, BN)
    rows = []
    for g in range(GG):
        sel = jnp.where(bb == g, h, 0.0)
        rows.append(jnp.max(sel, axis=1))
    blockmax = jnp.stack(rows, axis=0)
    acc_ref[...] = jnp.maximum(acc_ref[...], blockmax)

    @pl.when(i == NP // BN - 1)
    def _():
        out_ref[...] = jnp.dot(acc_ref[...], wl_ref[...]) + bl_ref[...]


def _pool(hT, batch3, wl, bl):
    grid = NP // BN
    return pl.pallas_call(
        _pool_body,
        grid=(grid,),
        in_specs=[
            pl.BlockSpec((CC, BN), lambda i: (0, i)),
            pl.BlockSpec((1, 1, BN), lambda i: (i, 0, 0)),
            pl.BlockSpec((CC, 1), lambda i: (0, 0)),
            pl.BlockSpec((1, 1), lambda i: (0, 0)),
        ],
        out_specs=pl.BlockSpec((GG, 1), lambda i: (0, 0)),
        out_shape=jax.ShapeDtypeStruct((GG, 1), jnp.float32),
        scratch_shapes=[pltpu.VMEM((GG, CC), jnp.float32)],
    )(hT, batch3, wl, bl)


def kernel(pos, edge_index, batch, W1a, b1a, W2a, b2a, W1b, b1b, W2b, b2b, Wl, bl):
    src = edge_index[0]
    dst = edge_index[1]
    posp = jnp.pad(pos, ((0, NP - NN), (0, 0)))
    batchp = jnp.pad(batch, (0, NP - NN), constant_values=GG)
    srcp = jnp.pad(src, (0, EP - EE))
    dstp = jnp.pad(dst, (0, EP - EE), constant_values=NN)

    A1, B1, B2 = _pre1(posp, W1a[:4] + W1a[4:], W1a[4:], W1b[32:],
                       b1a.reshape(1, CC))
    gA1, gB1 = _sc_gather(A1, B1, srcp, dstp)
    M1T = _mlp(gA1, gB1, W2a, b2a.reshape(CC, 1))
    h1T = _sc_scatter(M1T.reshape(CC * EP), dstp)

    A2 = _pre2(h1T, posp, W1b[:32], W1b[32:], b1b.reshape(1, CC))
    gA2, gB2 = _sc_gather(A2, B2, srcp, dstp)
    M2T = _mlp(gA2, gB2, W2b, b2b.reshape(CC, 1))
    h2T = _sc_scatter(M2T.reshape(CC * EP), dstp)

    out = _pool(h2T, batchp.reshape(NP // BN, 1, BN), Wl, bl.reshape(1, 1))
    return out


# speedup vs baseline: 1.5193x; 1.5193x over previous
import functools

import jax
import jax.numpy as jnp
from jax import lax
from jax.experimental import pallas as pl
from jax.experimental.pallas import tpu as pltpu
from jax.experimental.pallas import tpu_sc as plsc

NN = 100000
NP = 102400
EE = 1600000
EP = 1638400
GG = 64
CC = 32
NC, NS, LANES = 2, 16, 16
NW = NC * NS

BN = 2048
BE = 4096
GCH = 1024
SCH = 4096

_mesh = functools.partial(
    plsc.VectorSubcoreMesh,
    core_axis_name="c", subcore_axis_name="s",
    num_cores=NC, num_subcores=NS)


def _pre1_body(pos_ref, wA_ref, wB1_ref, wB2_ref, b_ref, a_ref, b1_ref, b2_ref):
    p = pos_ref[...]
    a_ref[...] = jnp.dot(p, wA_ref[...]) + b_ref[...]
    b1_ref[...] = jnp.dot(p, wB1_ref[...])
    b2_ref[...] = jnp.dot(p, wB2_ref[...])


def _pre1(pos, wA, wB1, wB2, b1a):
    grid = NP // BN
    return pl.pallas_call(
        _pre1_body,
        grid=(grid,),
        in_specs=[
            pl.BlockSpec((BN, 4), lambda i: (i, 0)),
            pl.BlockSpec((4, CC), lambda i: (0, 0)),
            pl.BlockSpec((4, CC), lambda i: (0, 0)),
            pl.BlockSpec((4, CC), lambda i: (0, 0)),
            pl.BlockSpec((1, CC), lambda i: (0, 0)),
        ],
        out_specs=[
            pl.BlockSpec((BN, CC), lambda i: (i, 0)),
            pl.BlockSpec((BN, CC), lambda i: (i, 0)),
            pl.BlockSpec((BN, CC), lambda i: (i, 0)),
        ],
        out_shape=[jax.ShapeDtypeStruct((NP, CC), jnp.float32)] * 3,
    )(pos, wA, wB1, wB2, b1a)


def _pre2_body(hT_ref, pos_ref, w32_ref, wd_ref, b_ref, a_ref):
    a = lax.dot_general(hT_ref[...], w32_ref[...], (((0,), (0,)), ((), ())))
    a_ref[...] = a + jnp.dot(pos_ref[...], wd_ref[...]) + b_ref[...]


def _pre2(hT, pos, w32, wd, b1b):
    grid = NP // BN
    return pl.pallas_call(
        _pre2_body,
        grid=(grid,),
        in_specs=[
            pl.BlockSpec((CC, BN), lambda i: (0, i)),
            pl.BlockSpec((BN, 4), lambda i: (i, 0)),
            pl.BlockSpec((CC, CC), lambda i: (0, 0)),
            pl.BlockSpec((4, CC), lambda i: (0, 0)),
            pl.BlockSpec((1, CC), lambda i: (0, 0)),
        ],
        out_specs=pl.BlockSpec((BN, CC), lambda i: (i, 0)),
        out_shape=jax.ShapeDtypeStruct((NP, CC), jnp.float32),
    )(hT, pos, w32, wd, b1b)


def _sc_gather_body(A_hbm, B_hbm, src_hbm, dst_hbm, gA_hbm, gB_hbm,
                    idxs_v, idxd_v, rowsA_v, rowsB_v, sem):
    wid = lax.axis_index("s") * NC + lax.axis_index("c")
    per_w = EP // NW
    base = wid * per_w

    def chunk(i, carry):
        off = base + i * GCH
        pltpu.sync_copy(src_hbm.at[pl.ds(off, GCH)], idxs_v)
        pltpu.sync_copy(dst_hbm.at[pl.ds(off, GCH)], idxd_v)
        cpA = pltpu.async_copy(A_hbm.at[idxs_v], rowsA_v, sem)
        cpB = pltpu.async_copy(B_hbm.at[idxd_v], rowsB_v, sem)
        cpA.wait()
        cpB.wait()
        pltpu.sync_copy(rowsA_v, gA_hbm.at[pl.ds(off, GCH)])
        pltpu.sync_copy(rowsB_v, gB_hbm.at[pl.ds(off, GCH)])
        return carry

    lax.fori_loop(0, per_w // GCH, chunk, 0)


def _sc_gather(A, B, src, dst):
    return pl.kernel(
        _sc_gather_body,
        compiler_params=pltpu.CompilerParams(use_tc_tiling_on_sc=False),
        out_type=[jax.ShapeDtypeStruct((EP, CC), jnp.float32),
                  jax.ShapeDtypeStruct((EP, CC), jnp.float32)],
        mesh=_mesh(),
        scratch_types=[
            pltpu.VMEM((GCH,), jnp.int32),
            pltpu.VMEM((GCH,), jnp.int32),
            pltpu.VMEM((GCH, CC), jnp.float32),
            pltpu.VMEM((GCH, CC), jnp.float32),
            pltpu.SemaphoreType.DMA,
        ],
    )(A, B, src, dst)


def _mlp_body(gA_ref, gB_ref, w2_ref, b2_ref, out_ref):
    pre = jnp.maximum(gA_ref[...] - gB_ref[...], 0.0)
    mt = lax.dot_general(w2_ref[...], pre, (((0,), (1,)), ((), ())))
    mt = mt + b2_ref[...]
    out_ref[...] = mt.reshape(CC, BE // 128, 128)


def _mlp(gA, gB, w2, b2):
    grid = EP // BE
    return pl.pallas_call(
        _mlp_body,
        grid=(grid,),
        in_specs=[
            pl.BlockSpec((BE, CC), lambda i: (i, 0)),
            pl.BlockSpec((BE, CC), lambda i: (i, 0)),
            pl.BlockSpec((CC, CC), lambda i: (0, 0)),
            pl.BlockSpec((CC, 1), lambda i: (0, 0)),
        ],
        out_specs=pl.BlockSpec((CC, BE // 128, 128), lambda i: (0, i, 0)),
        out_shape=jax.ShapeDtypeStruct((CC, EP // 128, 128), jnp.float32),
    )(gA, gB, w2, b2)


def _sc_scatter_body(mflat_hbm, dst_hbm, out_hbm, agg_v, midx_v, mval_v):
    wid = lax.axis_index("s") * NC + lax.axis_index("c")

    def zero(i, carry):
        agg_v[pl.ds(i * LANES, LANES)] = jnp.zeros((LANES,), jnp.float32)
        return carry

    lax.fori_loop(0, NP // LANES, zero, 0)

    def chunk(ci, carry):
        off = ci * SCH
        pltpu.sync_copy(dst_hbm.at[pl.ds(off, SCH)], midx_v)
        pltpu.sync_copy(mflat_hbm.at[pl.ds(wid * EP + off, SCH)], mval_v)

        iota = lax.iota(jnp.int32, LANES)

        def vec(v, cc):
            idx = midx_v[pl.ds(v * LANES, LANES)]
            val = mval_v[pl.ds(v * LANES, LANES)]
            k, w = plsc.sort_key_val(idx, val)
            for sh in (1, 2, 4, 8):
                prev = jnp.maximum(iota - sh, 0)
                kp = jnp.take(k, prev)
                wp = jnp.take(w, prev)
                w = jnp.maximum(w, jnp.where(kp == k, wp, -3.4e38))
            nxt = jnp.minimum(iota + 1, LANES - 1)
            last = (k != jnp.take(k, nxt)) | (iota == LANES - 1)
            cur = plsc.load_gather(agg_v, [k])
            plsc.store_scatter(agg_v, [k], jnp.maximum(cur, w), mask=last)
            return cc

        lax.fori_loop(0, SCH // LANES, vec, 0)
        return carry

    lax.fori_loop(0, EP // SCH, chunk, 0)
    pltpu.sync_copy(agg_v, out_hbm.at[wid])


def _sc_scatter(mflat, dst):
    return pl.kernel(
        _sc_scatter_body,
        compiler_params=pltpu.CompilerParams(use_tc_tiling_on_sc=False,
                                             needs_layout_passes=False),
        out_type=jax.ShapeDtypeStruct((CC, NP), jnp.float32),
        mesh=_mesh(),
        scratch_types=[
            pltpu.VMEM((NP,), jnp.float32),
            pltpu.VMEM((SCH,), jnp.int32),
            pltpu.VMEM((SCH,), jnp.float32),
        ],
    )(mflat, dst)


def _pool_body(hT_ref, batch_ref, wl_ref, bl_ref, out_ref, acc_ref):
    i = pl.program_id(0)

    @pl.when(i == 0)
    def _():
        acc_ref[...] = jnp.zeros_like(acc_ref)

    h = hT_ref[...]
    bb = batch_ref[...].reshape(1, BN)
    rows = []
    for g in range(GG):
        sel = jnp.where(bb == g, h, 0.0)
        rows.append(jnp.max(sel, axis=1))
    blockmax = jnp.stack(rows, axis=0)
    acc_ref[...] = jnp.maximum(acc_ref[...], blockmax)

    @pl.when(i == NP // BN - 1)
    def _():
        out_ref[...] = jnp.dot(acc_ref[...], wl_ref[...]) + bl_ref[...]


def _pool(hT, batch3, wl, bl):
    grid = NP // BN
    return pl.pallas_call(
        _pool_body,
        grid=(grid,),
        in_specs=[
            pl.BlockSpec((CC, BN), lambda i: (0, i)),
            pl.BlockSpec((1, 1, BN), lambda i: (i, 0, 0)),
            pl.BlockSpec((CC, 1), lambda i: (0, 0)),
            pl.BlockSpec((1, 1), lambda i: (0, 0)),
        ],
        out_specs=pl.BlockSpec((GG, 1), lambda i: (0, 0)),
        out_shape=jax.ShapeDtypeStruct((GG, 1), jnp.float32),
        scratch_shapes=[pltpu.VMEM((GG, CC), jnp.float32)],
    )(hT, batch3, wl, bl)


def kernel(pos, edge_index, batch, W1a, b1a, W2a, b2a, W1b, b1b, W2b, b2b, Wl, bl):
    src = edge_index[0]
    dst = edge_index[1]
    posp = jnp.pad(pos, ((0, NP - NN), (0, 0)))
    batchp = jnp.pad(batch, (0, NP - NN), constant_values=GG)
    srcp = jnp.pad(src, (0, EP - EE))
    dstp = jnp.pad(dst, (0, EP - EE), constant_values=NN)

    A1, B1, B2 = _pre1(posp, W1a[:4] + W1a[4:], W1a[4:], W1b[32:],
                       b1a.reshape(1, CC))
    gA1, gB1 = _sc_gather(A1, B1, srcp, dstp)
    M1T = _mlp(gA1, gB1, W2a, b2a.reshape(CC, 1))
    h1T = _sc_scatter(M1T.reshape(CC * EP), dstp)

    A2 = _pre2(h1T, posp, W1b[:32], W1b[32:], b1b.reshape(1, CC))
    gA2, gB2 = _sc_gather(A2, B2, srcp, dstp)
    M2T = _mlp(gA2, gB2, W2b, b2b.reshape(CC, 1))
    h2T = _sc_scatter(M2T.reshape(CC * EP), dstp)

    out = _pool(h2T, batchp.reshape(NP // BN, 1, BN), Wl, bl.reshape(1, 1))
    return out


# speedup vs baseline: 1.6730x; 1.1012x over previous
import functools

import jax
import jax.numpy as jnp
from jax import lax
from jax.experimental import pallas as pl
from jax.experimental.pallas import tpu as pltpu
from jax.experimental.pallas import tpu_sc as plsc

NN = 100000
NP = 102400
EE = 1600000
EP = 1638400
GG = 64
CC = 32
NC, NS, LANES = 2, 16, 16
NW = NC * NS

BN = 2048
BE = 4096
GCH = 1024
SCH = 4096

_mesh = functools.partial(
    plsc.VectorSubcoreMesh,
    core_axis_name="c", subcore_axis_name="s",
    num_cores=NC, num_subcores=NS)


def _pre1_body(pos_ref, wA_ref, wB1_ref, wB2_ref, b_ref, a_ref, b1_ref, b2_ref):
    p = pos_ref[...]
    a_ref[...] = jnp.dot(p, wA_ref[...]) + b_ref[...]
    b1_ref[...] = jnp.dot(p, wB1_ref[...])
    b2_ref[...] = jnp.dot(p, wB2_ref[...])


def _pre1(pos, wA, wB1, wB2, b1a):
    grid = NP // BN
    return pl.pallas_call(
        _pre1_body,
        grid=(grid,),
        in_specs=[
            pl.BlockSpec((BN, 4), lambda i: (i, 0)),
            pl.BlockSpec((4, CC), lambda i: (0, 0)),
            pl.BlockSpec((4, CC), lambda i: (0, 0)),
            pl.BlockSpec((4, CC), lambda i: (0, 0)),
            pl.BlockSpec((1, CC), lambda i: (0, 0)),
        ],
        out_specs=[
            pl.BlockSpec((BN, CC), lambda i: (i, 0)),
            pl.BlockSpec((BN, CC), lambda i: (i, 0)),
            pl.BlockSpec((BN, CC), lambda i: (i, 0)),
        ],
        out_shape=[jax.ShapeDtypeStruct((NP, CC), jnp.float32)] * 3,
    )(pos, wA, wB1, wB2, b1a)


def _pre2_body(hT_ref, pos_ref, w32_ref, wd_ref, b_ref, a_ref):
    a = lax.dot_general(hT_ref[...], w32_ref[...], (((0,), (0,)), ((), ())))
    a_ref[...] = a + jnp.dot(pos_ref[...], wd_ref[...]) + b_ref[...]


def _pre2(hT, pos, w32, wd, b1b):
    grid = NP // BN
    return pl.pallas_call(
        _pre2_body,
        grid=(grid,),
        in_specs=[
            pl.BlockSpec((CC, BN), lambda i: (0, i)),
            pl.BlockSpec((BN, 4), lambda i: (i, 0)),
            pl.BlockSpec((CC, CC), lambda i: (0, 0)),
            pl.BlockSpec((4, CC), lambda i: (0, 0)),
            pl.BlockSpec((1, CC), lambda i: (0, 0)),
        ],
        out_specs=pl.BlockSpec((BN, CC), lambda i: (i, 0)),
        out_shape=jax.ShapeDtypeStruct((NP, CC), jnp.float32),
    )(hT, pos, w32, wd, b1b)


def _sc_gather_body(A_hbm, B_hbm, src_hbm, dst_hbm, gA_hbm, gB_hbm,
                    idxs_v, idxd_v, rowsA_v, rowsB_v, sem):
    wid = lax.axis_index("s") * NC + lax.axis_index("c")
    per_w = EP // NW
    base = wid * per_w

    def chunk(i, carry):
        off = base + i * GCH
        pltpu.sync_copy(src_hbm.at[pl.ds(off, GCH)], idxs_v)
        pltpu.sync_copy(dst_hbm.at[pl.ds(off, GCH)], idxd_v)
        cpA = pltpu.async_copy(A_hbm.at[idxs_v], rowsA_v, sem)
        cpB = pltpu.async_copy(B_hbm.at[idxd_v], rowsB_v, sem)
        cpA.wait()
        cpB.wait()
        pltpu.sync_copy(rowsA_v, gA_hbm.at[pl.ds(off, GCH)])
        pltpu.sync_copy(rowsB_v, gB_hbm.at[pl.ds(off, GCH)])
        return carry

    lax.fori_loop(0, per_w // GCH, chunk, 0)


def _sc_gather(A, B, src, dst):
    return pl.kernel(
        _sc_gather_body,
        compiler_params=pltpu.CompilerParams(use_tc_tiling_on_sc=False),
        out_type=[jax.ShapeDtypeStruct((EP, CC), jnp.float32),
                  jax.ShapeDtypeStruct((EP, CC), jnp.float32)],
        mesh=_mesh(),
        scratch_types=[
            pltpu.VMEM((GCH,), jnp.int32),
            pltpu.VMEM((GCH,), jnp.int32),
            pltpu.VMEM((GCH, CC), jnp.float32),
            pltpu.VMEM((GCH, CC), jnp.float32),
            pltpu.SemaphoreType.DMA,
        ],
    )(A, B, src, dst)


def _mlp_body(gA_ref, gB_ref, w2_ref, b2_ref, out_ref):
    pre = jnp.maximum(gA_ref[...] - gB_ref[...], 0.0)
    mt = lax.dot_general(w2_ref[...], pre, (((0,), (1,)), ((), ())))
    mt = mt + b2_ref[...]
    out_ref[...] = mt.reshape(CC, BE // 128, 128)


def _mlp(gA, gB, w2, b2):
    grid = EP // BE
    return pl.pallas_call(
        _mlp_body,
        grid=(grid,),
        in_specs=[
            pl.BlockSpec((BE, CC), lambda i: (i, 0)),
            pl.BlockSpec((BE, CC), lambda i: (i, 0)),
            pl.BlockSpec((CC, CC), lambda i: (0, 0)),
            pl.BlockSpec((CC, 1), lambda i: (0, 0)),
        ],
        out_specs=pl.BlockSpec((CC, BE // 128, 128), lambda i: (0, i, 0)),
        out_shape=jax.ShapeDtypeStruct((CC, EP // 128, 128), jnp.float32),
    )(gA, gB, w2, b2)


def _sc_scatter_body(mflat_hbm, dst_hbm, out_hbm, agg_v, midx_v, mval_v, sem):
    wid = lax.axis_index("s") * NC + lax.axis_index("c")

    def zero(i, carry):
        agg_v[pl.ds(i * LANES, LANES)] = jnp.zeros((LANES,), jnp.float32)
        return carry

    lax.fori_loop(0, NP // LANES, zero, 0)

    iota = lax.iota(jnp.int32, LANES)
    nchunks = EP // SCH

    def start(ci, b):
        off = ci * SCH
        pltpu.async_copy(dst_hbm.at[pl.ds(off, SCH)], midx_v.at[b], sem)
        pltpu.async_copy(mflat_hbm.at[pl.ds(wid * EP + off, SCH)], mval_v.at[b], sem)

    def wait(ci, b):
        off = ci * SCH
        pltpu.make_async_copy(dst_hbm.at[pl.ds(off, SCH)], midx_v.at[b], sem).wait()
        pltpu.make_async_copy(mflat_hbm.at[pl.ds(wid * EP + off, SCH)], mval_v.at[b], sem).wait()

    start(0, 0)

    def chunk(ci, carry):
        b = lax.rem(ci, 2)
        wait(ci, b)
        nxt = jnp.minimum(ci + 1, nchunks - 1)
        start(nxt, 1 - b)
        midx_b = midx_v.at[b]
        mval_b = mval_v.at[b]

        def vec(v, cc):
            idx = midx_b[pl.ds(v * LANES, LANES)]
            val = mval_b[pl.ds(v * LANES, LANES)]
            k, w = plsc.sort_key_val(idx, val)
            for sh in (1, 2, 4, 8):
                prev = jnp.maximum(iota - sh, 0)
                kp = jnp.take(k, prev)
                wp = jnp.take(w, prev)
                w = jnp.maximum(w, jnp.where(kp == k, wp, -3.4e38))
            nxt = jnp.minimum(iota + 1, LANES - 1)
            last = (k != jnp.take(k, nxt)) | (iota == LANES - 1)
            cur = plsc.load_gather(agg_v, [k])
            plsc.store_scatter(agg_v, [k], jnp.maximum(cur, w), mask=last)
            return cc

        lax.fori_loop(0, SCH // LANES, vec, 0)
        return carry

    lax.fori_loop(0, nchunks, chunk, 0)
    wait(nchunks - 1, lax.rem(nchunks, 2))
    pltpu.sync_copy(agg_v, out_hbm.at[wid])


def _sc_scatter(mflat, dst):
    return pl.kernel(
        _sc_scatter_body,
        compiler_params=pltpu.CompilerParams(use_tc_tiling_on_sc=False,
                                             needs_layout_passes=False),
        out_type=jax.ShapeDtypeStruct((CC, NP), jnp.float32),
        mesh=_mesh(),
        scratch_types=[
            pltpu.VMEM((NP,), jnp.float32),
            pltpu.VMEM((2, SCH), jnp.int32),
            pltpu.VMEM((2, SCH), jnp.float32),
            pltpu.SemaphoreType.DMA,
        ],
    )(mflat, dst)


def _pool_body(hT_ref, batch_ref, wl_ref, bl_ref, out_ref, acc_ref):
    i = pl.program_id(0)

    @pl.when(i == 0)
    def _():
        acc_ref[...] = jnp.zeros_like(acc_ref)

    h = hT_ref[...]
    bb = batch_ref[...].reshape(1, BN)
    rows = []
    for g in range(GG):
        sel = jnp.where(bb == g, h, 0.0)
        rows.append(jnp.max(sel, axis=1))
    blockmax = jnp.stack(rows, axis=0)
    acc_ref[...] = jnp.maximum(acc_ref[...], blockmax)

    @pl.when(i == NP // BN - 1)
    def _():
        out_ref[...] = jnp.dot(acc_ref[...], wl_ref[...]) + bl_ref[...]


def _pool(hT, batch3, wl, bl):
    grid = NP // BN
    return pl.pallas_call(
        _pool_body,
        grid=(grid,),
        in_specs=[
            pl.BlockSpec((CC, BN), lambda i: (0, i)),
            pl.BlockSpec((1, 1, BN), lambda i: (i, 0, 0)),
            pl.BlockSpec((CC, 1), lambda i: (0, 0)),
            pl.BlockSpec((1, 1), lambda i: (0, 0)),
        ],
        out_specs=pl.BlockSpec((GG, 1), lambda i: (0, 0)),
        out_shape=jax.ShapeDtypeStruct((GG, 1), jnp.float32),
        scratch_shapes=[pltpu.VMEM((GG, CC), jnp.float32)],
    )(hT, batch3, wl, bl)


def kernel(pos, edge_index, batch, W1a, b1a, W2a, b2a, W1b, b1b, W2b, b2b, Wl, bl):
    src = edge_index[0]
    dst = edge_index[1]
    posp = jnp.pad(pos, ((0, NP - NN), (0, 0)))
    batchp = jnp.pad(batch, (0, NP - NN), constant_values=GG)
    srcp = jnp.pad(src, (0, EP - EE))
    dstp = jnp.pad(dst, (0, EP - EE), constant_values=NN)

    A1, B1, B2 = _pre1(posp, W1a[:4] + W1a[4:], W1a[4:], W1b[32:],
                       b1a.reshape(1, CC))
    gA1, gB1 = _sc_gather(A1, B1, srcp, dstp)
    M1T = _mlp(gA1, gB1, W2a, b2a.reshape(CC, 1))
    h1T = _sc_scatter(M1T.reshape(CC * EP), dstp)

    A2 = _pre2(h1T, posp, W1b[:32], W1b[32:], b1b.reshape(1, CC))
    gA2, gB2 = _sc_gather(A2, B2, srcp, dstp)
    M2T = _mlp(gA2, gB2, W2b, b2b.reshape(CC, 1))
    h2T = _sc_scatter(M2T.reshape(CC * EP), dstp)

    out = _pool(h2T, batchp.reshape(NP // BN, 1, BN), Wl, bl.reshape(1, 1))
    return out


# speedup vs baseline: 2.0708x; 1.2378x over previous
import functools

import jax
import jax.numpy as jnp
from jax import lax
from jax.experimental import pallas as pl
from jax.experimental.pallas import tpu as pltpu
from jax.experimental.pallas import tpu_sc as plsc

NN = 100000
NP = 102400
EE = 1600000
EP = 1638400
GG = 64
CC = 32
NC, NS, LANES = 2, 16, 16
NW = NC * NS

BN = 2048
BE = 4096
GCH = 1024
SCH = 4096

_mesh = functools.partial(
    plsc.VectorSubcoreMesh,
    core_axis_name="c", subcore_axis_name="s",
    num_cores=NC, num_subcores=NS)


def _pre1_body(pos_ref, wA_ref, wB1_ref, wB2_ref, b_ref, a_ref, b1_ref, b2_ref):
    p = pos_ref[...]
    a_ref[...] = jnp.dot(p, wA_ref[...]) + b_ref[...]
    b1_ref[...] = jnp.dot(p, wB1_ref[...])
    b2_ref[...] = jnp.dot(p, wB2_ref[...])


def _pre1(pos, wA, wB1, wB2, b1a):
    grid = NP // BN
    return pl.pallas_call(
        _pre1_body,
        grid=(grid,),
        in_specs=[
            pl.BlockSpec((BN, 4), lambda i: (i, 0)),
            pl.BlockSpec((4, CC), lambda i: (0, 0)),
            pl.BlockSpec((4, CC), lambda i: (0, 0)),
            pl.BlockSpec((4, CC), lambda i: (0, 0)),
            pl.BlockSpec((1, CC), lambda i: (0, 0)),
        ],
        out_specs=[
            pl.BlockSpec((BN, CC), lambda i: (i, 0)),
            pl.BlockSpec((BN, CC), lambda i: (i, 0)),
            pl.BlockSpec((BN, CC), lambda i: (i, 0)),
        ],
        out_shape=[jax.ShapeDtypeStruct((NP, CC), jnp.float32)] * 3,
    )(pos, wA, wB1, wB2, b1a)


def _pre2_body(hT_ref, pos_ref, w32_ref, wd_ref, b_ref, a_ref):
    a = lax.dot_general(hT_ref[...], w32_ref[...], (((0,), (0,)), ((), ())))
    a_ref[...] = a + jnp.dot(pos_ref[...], wd_ref[...]) + b_ref[...]


def _pre2(hT, pos, w32, wd, b1b):
    grid = NP // BN
    return pl.pallas_call(
        _pre2_body,
        grid=(grid,),
        in_specs=[
            pl.BlockSpec((CC, BN), lambda i: (0, i)),
            pl.BlockSpec((BN, 4), lambda i: (i, 0)),
            pl.BlockSpec((CC, CC), lambda i: (0, 0)),
            pl.BlockSpec((4, CC), lambda i: (0, 0)),
            pl.BlockSpec((1, CC), lambda i: (0, 0)),
        ],
        out_specs=pl.BlockSpec((BN, CC), lambda i: (i, 0)),
        out_shape=jax.ShapeDtypeStruct((NP, CC), jnp.float32),
    )(hT, pos, w32, wd, b1b)


def _sc_gather_body(A_hbm, B_hbm, src_hbm, dst_hbm, gA_hbm, gB_hbm,
                    idxs_v, idxd_v, rowsA_v, rowsB_v, sem):
    wid = lax.axis_index("s") * NC + lax.axis_index("c")
    per_w = EP // NW
    base = wid * per_w

    def chunk(i, carry):
        off = base + i * GCH
        pltpu.sync_copy(src_hbm.at[pl.ds(off, GCH)], idxs_v)
        pltpu.sync_copy(dst_hbm.at[pl.ds(off, GCH)], idxd_v)
        cpA = pltpu.async_copy(A_hbm.at[idxs_v], rowsA_v, sem)
        cpB = pltpu.async_copy(B_hbm.at[idxd_v], rowsB_v, sem)
        cpA.wait()
        cpB.wait()
        pltpu.sync_copy(rowsA_v, gA_hbm.at[pl.ds(off, GCH)])
        pltpu.sync_copy(rowsB_v, gB_hbm.at[pl.ds(off, GCH)])
        return carry

    lax.fori_loop(0, per_w // GCH, chunk, 0)


def _sc_gather(A, B, src, dst):
    return pl.kernel(
        _sc_gather_body,
        compiler_params=pltpu.CompilerParams(use_tc_tiling_on_sc=False),
        out_type=[jax.ShapeDtypeStruct((EP, CC), jnp.float32),
                  jax.ShapeDtypeStruct((EP, CC), jnp.float32)],
        mesh=_mesh(),
        scratch_types=[
            pltpu.VMEM((GCH,), jnp.int32),
            pltpu.VMEM((GCH,), jnp.int32),
            pltpu.VMEM((GCH, CC), jnp.float32),
            pltpu.VMEM((GCH, CC), jnp.float32),
            pltpu.SemaphoreType.DMA,
        ],
    )(A, B, src, dst)


def _mlp_body(gA_ref, gB_ref, w4_ref, b4_ref, out_ref):
    pre = jnp.maximum(gA_ref[...] - gB_ref[...], 0.0)
    q = lax.dot_general(w4_ref[...], pre, (((0,), (1,)), ((), ())))
    q = q + b4_ref[...]
    out_ref[...] = q.reshape(4 * CC, BE // 4 // 128, 128)


def _mlp(gA4, gB4, w4, b4):
    grid = EP // BE
    return pl.pallas_call(
        _mlp_body,
        grid=(grid,),
        in_specs=[
            pl.BlockSpec((BE // 4, 128), lambda i: (i, 0)),
            pl.BlockSpec((BE // 4, 128), lambda i: (i, 0)),
            pl.BlockSpec((4 * CC, 4 * CC), lambda i: (0, 0)),
            pl.BlockSpec((4 * CC, 1), lambda i: (0, 0)),
        ],
        out_specs=pl.BlockSpec((4 * CC, BE // 4 // 128, 128), lambda i: (0, i, 0)),
        out_shape=jax.ShapeDtypeStruct((4 * CC, EP // 4 // 128, 128), jnp.float32),
    )(gA4, gB4, w4, b4)


def _sc_scatter_body(mflat_hbm, dst_hbm, out_hbm, agg_v, midx_v, mval_v, sem):
    wid = lax.axis_index("s") * NC + lax.axis_index("c")

    def zero(i, carry):
        agg_v[pl.ds(i * LANES, LANES)] = jnp.zeros((LANES,), jnp.float32)
        return carry

    lax.fori_loop(0, NP // LANES, zero, 0)

    iota = lax.iota(jnp.int32, LANES)
    nchunks = EP // SCH

    sub = EP // 4
    csub = sub // SCH

    def offs(ci):
        j = ci // csub
        c2 = lax.rem(ci, csub)
        return j * sub + c2 * SCH, (j * CC + wid) * sub + c2 * SCH

    def start(ci, b):
        ioff, voff = offs(ci)
        pltpu.async_copy(dst_hbm.at[pl.ds(ioff, SCH)], midx_v.at[b], sem)
        pltpu.async_copy(mflat_hbm.at[pl.ds(voff, SCH)], mval_v.at[b], sem)

    def wait(ci, b):
        ioff, voff = offs(ci)
        pltpu.make_async_copy(dst_hbm.at[pl.ds(ioff, SCH)], midx_v.at[b], sem).wait()
        pltpu.make_async_copy(mflat_hbm.at[pl.ds(voff, SCH)], mval_v.at[b], sem).wait()

    start(0, 0)

    def chunk(ci, carry):
        b = lax.rem(ci, 2)
        wait(ci, b)
        nxt = jnp.minimum(ci + 1, nchunks - 1)
        start(nxt, 1 - b)
        midx_b = midx_v.at[b]
        mval_b = mval_v.at[b]

        def vec(v, cc):
            idx = midx_b[pl.ds(v * LANES, LANES)]
            val = mval_b[pl.ds(v * LANES, LANES)]
            k, w = plsc.sort_key_val(idx, val)
            for sh in (1, 2, 4, 8):
                prev = jnp.maximum(iota - sh, 0)
                kp = jnp.take(k, prev)
                wp = jnp.take(w, prev)
                w = jnp.maximum(w, jnp.where(kp == k, wp, -3.4e38))
            nxt = jnp.minimum(iota + 1, LANES - 1)
            last = (k != jnp.take(k, nxt)) | (iota == LANES - 1)
            cur = plsc.load_gather(agg_v, [k])
            plsc.store_scatter(agg_v, [k], jnp.maximum(cur, w), mask=last)
            return cc

        lax.fori_loop(0, SCH // LANES, vec, 0)
        return carry

    lax.fori_loop(0, nchunks, chunk, 0)
    wait(nchunks - 1, lax.rem(nchunks, 2))
    pltpu.sync_copy(agg_v, out_hbm.at[wid])


def _sc_scatter(mflat, dst):
    return pl.kernel(
        _sc_scatter_body,
        compiler_params=pltpu.CompilerParams(use_tc_tiling_on_sc=False,
                                             needs_layout_passes=False),
        out_type=jax.ShapeDtypeStruct((CC, NP), jnp.float32),
        mesh=_mesh(),
        scratch_types=[
            pltpu.VMEM((NP,), jnp.float32),
            pltpu.VMEM((2, SCH), jnp.int32),
            pltpu.VMEM((2, SCH), jnp.float32),
            pltpu.SemaphoreType.DMA,
        ],
    )(mflat, dst)


def _pool_body(hT_ref, batch_ref, wl_ref, bl_ref, out_ref, acc_ref):
    i = pl.program_id(0)

    @pl.when(i == 0)
    def _():
        acc_ref[...] = jnp.zeros_like(acc_ref)

    h = hT_ref[...]
    bb = batch_ref[...].reshape(1, BN)
    rows = []
    for g in range(GG):
        sel = jnp.where(bb == g, h, 0.0)
        rows.append(jnp.max(sel, axis=1))
    blockmax = jnp.stack(rows, axis=0)
    acc_ref[...] = jnp.maximum(acc_ref[...], blockmax)

    @pl.when(i == NP // BN - 1)
    def _():
        out_ref[...] = jnp.dot(acc_ref[...], wl_ref[...]) + bl_ref[...]


def _pool(hT, batch3, wl, bl):
    grid = NP // BN
    return pl.pallas_call(
        _pool_body,
        grid=(grid,),
        in_specs=[
            pl.BlockSpec((CC, BN), lambda i: (0, i)),
            pl.BlockSpec((1, 1, BN), lambda i: (i, 0, 0)),
            pl.BlockSpec((CC, 1), lambda i: (0, 0)),
            pl.BlockSpec((1, 1), lambda i: (0, 0)),
        ],
        out_specs=pl.BlockSpec((GG, 1), lambda i: (0, 0)),
        out_shape=jax.ShapeDtypeStruct((GG, 1), jnp.float32),
        scratch_shapes=[pltpu.VMEM((GG, CC), jnp.float32)],
    )(hT, batch3, wl, bl)


def kernel(pos, edge_index, batch, W1a, b1a, W2a, b2a, W1b, b1b, W2b, b2b, Wl, bl):
    src = edge_index[0]
    dst = edge_index[1]
    posp = jnp.pad(pos, ((0, NP - NN), (0, 0)))
    batchp = jnp.pad(batch, (0, NP - NN), constant_values=GG)
    srcp = jnp.pad(src, (0, EP - EE))
    dstp = jnp.pad(dst, (0, EP - EE), constant_values=NN)
    dstj = dstp.reshape(EP // 4, 4).T.reshape(EP)
    z = jnp.zeros((CC, CC), jnp.float32)
    w4a = jnp.block([[W2a, z, z, z], [z, W2a, z, z], [z, z, W2a, z], [z, z, z, W2a]])
    w4b = jnp.block([[W2b, z, z, z], [z, W2b, z, z], [z, z, W2b, z], [z, z, z, W2b]])
    b4a = jnp.tile(b2a, 4).reshape(4 * CC, 1)
    b4b = jnp.tile(b2b, 4).reshape(4 * CC, 1)

    A1, B1, B2 = _pre1(posp, W1a[:4] + W1a[4:], W1a[4:], W1b[32:],
                       b1a.reshape(1, CC))
    gA1, gB1 = _sc_gather(A1, B1, srcp, dstp)
    M1T = _mlp(gA1.reshape(EP // 4, 128), gB1.reshape(EP // 4, 128), w4a, b4a)
    h1T = _sc_scatter(M1T.reshape(CC * EP), dstj)

    A2 = _pre2(h1T, posp, W1b[:32], W1b[32:], b1b.reshape(1, CC))
    gA2, gB2 = _sc_gather(A2, B2, srcp, dstp)
    M2T = _mlp(gA2.reshape(EP // 4, 128), gB2.reshape(EP // 4, 128), w4b, b4b)
    h2T = _sc_scatter(M2T.reshape(CC * EP), dstj)

    out = _pool(h2T, batchp.reshape(NP // BN, 1, BN), Wl, bl.reshape(1, 1))
    return out
